# baseline (speedup 1.0000x reference)
import jax
import jax.numpy as jnp
from jax.experimental import pallas as pl
from jax.experimental.pallas import tpu as pltpu

N_BATCH = 16
WIDTH = 256
T_LEN = 1024
N_CODES = 1024
TOTAL_ROWS = N_BATCH * T_LEN


def _vq_body(z_ref, cbm2_ref, cbt_ref, out_ref, loss_ref, perp_ref,
             counts_acc, loss_acc, cbsq_acc):
    n = pl.program_id(0)

    @pl.when(n == 0)
    def _init():
        counts_acc[...] = jnp.zeros_like(counts_acc)
        loss_acc[...] = jnp.zeros_like(loss_acc)
        cbm2 = cbm2_ref[...]
        cbsq_acc[...] = jnp.sum(cbm2 * cbm2, axis=1, keepdims=True) * 0.25

    zb = z_ref[0]

    dotm2 = jnp.dot(cbm2_ref[...], zb,
                    preferred_element_type=jnp.float32)
    z_sq = jnp.sum(zb * zb, axis=0, keepdims=True)
    d = (z_sq + cbsq_acc[...]) + dotm2

    minval = jnp.min(d, axis=0, keepdims=True)
    kiota = jax.lax.broadcasted_iota(jnp.int32, d.shape, 0)
    cand = jnp.where(d == minval, kiota, jnp.int32(N_CODES))
    idx = jnp.min(cand, axis=0, keepdims=True)
    onehot = jnp.where(kiota == idx, jnp.float32(1.0),
                       jnp.float32(0.0)).astype(jnp.bfloat16)

    zq = jnp.dot(cbt_ref[...], onehot,
                 preferred_element_type=jnp.float32)
    out_ref[0] = zq

    diff = zq - zb
    loss_acc[...] += jnp.sum(diff * diff)[None, None]
    ones_t = jnp.ones((T_LEN, 1), jnp.bfloat16)
    counts_acc[...] += jnp.dot(onehot, ones_t,
                               preferred_element_type=jnp.float32)

    @pl.when(n == N_BATCH - 1)
    def _finalize():
        scale = jnp.float32(1.25 / (TOTAL_ROWS * WIDTH))
        loss_ref[...] = loss_acc[...] * scale
        e_mean = counts_acc[...] * jnp.float32(1.0 / TOTAL_ROWS)
        ent = jnp.sum(e_mean * jnp.log(e_mean + jnp.float32(1e-10)))
        perp_ref[...] = jnp.exp(-ent)[None, None]


def kernel(z, codebook, W_map, b_map):
    del W_map, b_map
    cbm2 = -2.0 * codebook
    cbt = codebook.T.astype(jnp.bfloat16)

    out_shapes = (
        jax.ShapeDtypeStruct((N_BATCH, WIDTH, T_LEN), jnp.float32),
        jax.ShapeDtypeStruct((1, 1), jnp.float32),
        jax.ShapeDtypeStruct((1, 1), jnp.float32),
    )
    z_q_out, loss, perp = pl.pallas_call(
        _vq_body,
        grid=(N_BATCH,),
        in_specs=[
            pl.BlockSpec((1, WIDTH, T_LEN), lambda n: (n, 0, 0)),
            pl.BlockSpec((N_CODES, WIDTH), lambda n: (0, 0)),
            pl.BlockSpec((WIDTH, N_CODES), lambda n: (0, 0)),
        ],
        out_specs=(
            pl.BlockSpec((1, WIDTH, T_LEN), lambda n: (n, 0, 0)),
            pl.BlockSpec((1, 1), lambda n: (0, 0)),
            pl.BlockSpec((1, 1), lambda n: (0, 0)),
        ),
        scratch_shapes=[
            pltpu.VMEM((N_CODES, 1), jnp.float32),
            pltpu.VMEM((1, 1), jnp.float32),
            pltpu.VMEM((N_CODES, 1), jnp.float32),
        ],
        out_shape=out_shapes,
    )(z, cbm2, cbt)
    return (z_q_out, loss[0, 0], perp[0, 0])


# speedup vs baseline: 1.1242x; 1.1242x over previous
import jax
import jax.numpy as jnp
from jax.experimental import pallas as pl
from jax.experimental.pallas import tpu as pltpu

N_BATCH = 16
WIDTH = 256
T_LEN = 1024
N_CODES = 1024
TOTAL_ROWS = N_BATCH * T_LEN


def _vq_body(z_ref, cbm2_ref, cbt_ref, out_ref, loss_ref, perp_ref,
             counts_acc, loss_acc, cbsq_acc, kiotaf_acc):
    n = pl.program_id(0)

    @pl.when(n == 0)
    def _init():
        counts_acc[...] = jnp.zeros_like(counts_acc)
        loss_acc[...] = jnp.zeros_like(loss_acc)
        cbm2 = cbm2_ref[...]
        cbsq_acc[...] = jnp.sum(cbm2 * cbm2, axis=1, keepdims=True) * 0.25
        kiotaf_acc[...] = jax.lax.broadcasted_iota(
            jnp.int32, (N_CODES, T_LEN), 0).astype(jnp.float32)

    zb = z_ref[0]

    dotm2 = jnp.dot(cbm2_ref[...], zb,
                    preferred_element_type=jnp.float32)
    z_sq = jnp.sum(zb * zb, axis=0, keepdims=True)
    d = (z_sq + cbsq_acc[...]) + dotm2

    minval = jnp.min(d, axis=0, keepdims=True)
    kf = kiotaf_acc[...]
    cand = jnp.where(d == minval, kf, jnp.float32(2 * N_CODES))
    idxf = jnp.min(cand, axis=0, keepdims=True)
    onehot = jnp.where(kf == idxf, jnp.float32(1.0),
                       jnp.float32(0.0))

    zq = jnp.dot(cbt_ref[...], onehot,
                 preferred_element_type=jnp.float32)
    out_ref[0] = zq

    loss_acc[...] += jnp.sum(minval)[None, None]
    counts_acc[...] += jnp.sum(onehot, axis=1, keepdims=True)

    @pl.when(n == N_BATCH - 1)
    def _finalize():
        scale = jnp.float32(1.25 / (TOTAL_ROWS * WIDTH))
        loss_ref[...] = loss_acc[...] * scale
        e_mean = counts_acc[...] * jnp.float32(1.0 / TOTAL_ROWS)
        ent = jnp.sum(e_mean * jnp.log(e_mean + jnp.float32(1e-10)))
        perp_ref[...] = jnp.exp(-ent)[None, None]


def kernel(z, codebook, W_map, b_map):
    del W_map, b_map
    cbm2 = -2.0 * codebook
    cbt = codebook.T

    out_shapes = (
        jax.ShapeDtypeStruct((N_BATCH, WIDTH, T_LEN), jnp.float32),
        jax.ShapeDtypeStruct((1, 1), jnp.float32),
        jax.ShapeDtypeStruct((1, 1), jnp.float32),
    )
    z_q_out, loss, perp = pl.pallas_call(
        _vq_body,
        grid=(N_BATCH,),
        in_specs=[
            pl.BlockSpec((1, WIDTH, T_LEN), lambda n: (n, 0, 0)),
            pl.BlockSpec((N_CODES, WIDTH), lambda n: (0, 0)),
            pl.BlockSpec((WIDTH, N_CODES), lambda n: (0, 0)),
        ],
        out_specs=(
            pl.BlockSpec((1, WIDTH, T_LEN), lambda n: (n, 0, 0)),
            pl.BlockSpec((1, 1), lambda n: (0, 0)),
            pl.BlockSpec((1, 1), lambda n: (0, 0)),
        ),
        scratch_shapes=[
            pltpu.VMEM((N_CODES, 1), jnp.float32),
            pltpu.VMEM((1, 1), jnp.float32),
            pltpu.VMEM((N_CODES, 1), jnp.float32),
            pltpu.VMEM((N_CODES, T_LEN), jnp.float32),
        ],
        out_shape=out_shapes,
    )(z, cbm2, cbt)
    return (z_q_out, loss[0, 0], perp[0, 0])


# speedup vs baseline: 1.2638x; 1.1242x over previous
import jax
import jax.numpy as jnp
from jax.experimental import pallas as pl
from jax.experimental.pallas import tpu as pltpu

N_BATCH = 16
WIDTH = 256
T_LEN = 1024
N_CODES = 1024
TOTAL_ROWS = N_BATCH * T_LEN


def _vq_body(z_ref, cb_ref, out_ref, loss_ref, perp_ref,
             counts_acc, loss_acc, cbm2_acc, cbt_acc, cbsq_acc, kiotaf_acc):
    n = pl.program_id(0)

    @pl.when(n == 0)
    def _init():
        counts_acc[...] = jnp.zeros_like(counts_acc)
        loss_acc[...] = jnp.zeros_like(loss_acc)
        cb = cb_ref[...]
        cbm2_acc[...] = -2.0 * cb
        cbt_acc[...] = cb.T
        cbsq_acc[...] = jnp.sum(cb * cb, axis=1, keepdims=True)
        kiotaf_acc[...] = jax.lax.broadcasted_iota(
            jnp.int32, (N_CODES, T_LEN), 0).astype(jnp.float32)

    zb = z_ref[0]

    dotm2 = jnp.dot(cbm2_acc[...], zb,
                    preferred_element_type=jnp.float32)
    z_sq = jnp.sum(zb * zb, axis=0, keepdims=True)
    d = (z_sq + cbsq_acc[...]) + dotm2

    minval = jnp.min(d, axis=0, keepdims=True)
    kf = kiotaf_acc[...]
    cand = jnp.where(d == minval, kf, jnp.float32(2 * N_CODES))
    idxf = jnp.min(cand, axis=0, keepdims=True)
    onehot = jnp.where(kf == idxf, jnp.float32(1.0),
                       jnp.float32(0.0))

    zq = jnp.dot(cbt_acc[...], onehot,
                 preferred_element_type=jnp.float32)
    out_ref[0] = zq

    loss_acc[...] += jnp.sum(minval)[None, None]
    counts_acc[...] += jnp.sum(onehot, axis=1, keepdims=True)

    @pl.when(n == N_BATCH - 1)
    def _finalize():
        scale = jnp.float32(1.25 / (TOTAL_ROWS * WIDTH))
        loss_ref[...] = loss_acc[...] * scale
        e_mean = counts_acc[...] * jnp.float32(1.0 / TOTAL_ROWS)
        ent = jnp.sum(e_mean * jnp.log(e_mean + jnp.float32(1e-10)))
        perp_ref[...] = jnp.exp(-ent)[None, None]


def kernel(z, codebook, W_map, b_map):
    del W_map, b_map
    out_shapes = (
        jax.ShapeDtypeStruct((N_BATCH, WIDTH, T_LEN), jnp.float32),
        jax.ShapeDtypeStruct((1, 1), jnp.float32),
        jax.ShapeDtypeStruct((1, 1), jnp.float32),
    )
    z_q_out, loss, perp = pl.pallas_call(
        _vq_body,
        grid=(N_BATCH,),
        in_specs=[
            pl.BlockSpec((1, WIDTH, T_LEN), lambda n: (n, 0, 0)),
            pl.BlockSpec((N_CODES, WIDTH), lambda n: (0, 0)),
        ],
        out_specs=(
            pl.BlockSpec((1, WIDTH, T_LEN), lambda n: (n, 0, 0)),
            pl.BlockSpec((1, 1), lambda n: (0, 0)),
            pl.BlockSpec((1, 1), lambda n: (0, 0)),
        ),
        scratch_shapes=[
            pltpu.VMEM((N_CODES, 1), jnp.float32),
            pltpu.VMEM((1, 1), jnp.float32),
            pltpu.VMEM((N_CODES, WIDTH), jnp.float32),
            pltpu.VMEM((WIDTH, N_CODES), jnp.float32),
            pltpu.VMEM((N_CODES, 1), jnp.float32),
            pltpu.VMEM((N_CODES, T_LEN), jnp.float32),
        ],
        out_shape=out_shapes,
    )(z, codebook)
    return (z_q_out, loss[0, 0], perp[0, 0])
